# baseline (speedup 1.0000x reference)
import functools

import jax
import jax.numpy as jnp
from jax import lax
from jax.experimental import pallas as pl
from jax.experimental.pallas import tpu as pltpu
from jax.experimental.pallas import tpu_sc as plsc

_B, _L, _D = 16384, 200, 16
_LD = _L * _D
_CB = 16


def _sc_body(x_hbm, pos_hbm, out_hbm, x_v, pos_v, out_v):
    info = plsc.get_sparse_core_info()
    nw = info.num_cores * info.num_subcores
    wid = lax.axis_index("s") * info.num_cores + lax.axis_index("c")
    rows = _B // nw
    pltpu.sync_copy(pos_hbm, pos_v)
    base0 = wid * rows

    def per_row(i, carry2):
        for loff in range(0, _L - 8, 16):
            x16 = x_v[i, pl.ds(loff, 16)]
            for j in range(16):
                off = (loff + j) * _D
                out_v[i, pl.ds(off, _D)] = x16[j] + pos_v[pl.ds(off, _D)]
        x16 = x_v[i, pl.ds(_L - 16, 16)]
        for j in range(8, 16):
            off = (_L - 16 + j) * _D
            out_v[i, pl.ds(off, _D)] = x16[j] + pos_v[pl.ds(off, _D)]
        return carry2

    def chunk(c, carry):
        base = base0 + c * _CB
        pltpu.sync_copy(x_hbm.at[pl.ds(base, _CB)], x_v)
        lax.fori_loop(0, _CB, per_row, 0)
        pltpu.sync_copy(out_v, out_hbm.at[pl.ds(base, _CB)])
        return carry

    lax.fori_loop(0, rows // _CB, chunk, 0)


def kernel(x, pos_table):
    B, L = x.shape
    D = pos_table.shape[-1]
    pos_flat = pos_table.reshape(L * D)
    k = functools.partial(
        pl.kernel,
        mesh=plsc.VectorSubcoreMesh(core_axis_name="c", subcore_axis_name="s"),
        out_type=jax.ShapeDtypeStruct((B, L * D), x.dtype),
        scratch_types=[
            pltpu.VMEM((_CB, L), jnp.float32),
            pltpu.VMEM((L * D,), jnp.float32),
            pltpu.VMEM((_CB, L * D), jnp.float32),
        ],
    )(_sc_body)
    y = k(x, pos_flat)
    return y.reshape(B, L, D)


# speedup vs baseline: 10.9295x; 10.9295x over previous
import jax
import jax.numpy as jnp
from jax.experimental import pallas as pl

_RBX = 8


def _body(xt_ref, pos_ref, o_ref):
    nx, nb = xt_ref.shape
    nr = o_ref.shape[0]
    d = nr // nx
    xt = xt_ref[...]
    y = jnp.broadcast_to(xt[:, None, :], (nx, d, nb)).reshape(nr, nb)
    o_ref[...] = y + jnp.broadcast_to(pos_ref[...], (nr, nb))


def kernel(x, pos_table):
    B, L = x.shape
    D = pos_table.shape[-1]
    xt = x.T
    pos_col = pos_table.reshape(L * D, 1)
    rb = _RBX * D
    y = pl.pallas_call(
        _body,
        grid=(L // _RBX,),
        in_specs=[
            pl.BlockSpec((_RBX, B), lambda i: (i, 0)),
            pl.BlockSpec((rb, 1), lambda i: (i, 0)),
        ],
        out_specs=pl.BlockSpec((rb, B), lambda i: (i, 0)),
        out_shape=jax.ShapeDtypeStruct((L * D, B), x.dtype),
    )(xt, pos_col)
    return y.reshape(L, D, B).transpose(2, 0, 1)
